# baseline (speedup 1.0000x reference)
import functools

import jax
import jax.numpy as jnp
from jax import lax
from jax.experimental import pallas as pl
from jax.experimental.pallas import tpu as pltpu
from jax.experimental.pallas import tpu_sc as plsc

NC = 2
NS = 16
NW = NC * NS
CHUNK = 128
NBUF = 8


def _mesh():
  return plsc.VectorSubcoreMesh(
      core_axis_name="c", subcore_axis_name="s", num_cores=NC,
      num_subcores=NS)


def _hist_kernel(nacc, nch):
  rpt = nacc // NS

  @functools.partial(
      pl.kernel,
      out_type=jax.ShapeDtypeStruct((NC * nacc,), jnp.float32),
      mesh=_mesh(),
      scratch_types=[
          pltpu.VMEM((nch, CHUNK), jnp.int32),
          pltpu.VMEM((CHUNK,), jnp.float32),
          pltpu.VMEM((rpt,), jnp.float32),
          pltpu.VMEM_SHARED((nacc,), jnp.float32),
          pltpu.SemaphoreType.DMA,
      ],
  )
  def k(rows_hbm, zeros_hbm, out_hbm, ridx_v, ones_v, zbuf_v, acc, sem):
    c = lax.axis_index("c")
    s = lax.axis_index("s")
    wid = c * NS + s
    for i in range(CHUNK // 16):
      ones_v[pl.ds(16 * i, 16)] = jnp.ones((16,), jnp.float32)
    pltpu.sync_copy(zeros_hbm.at[pl.ds(s * rpt, rpt)], zbuf_v)
    pltpu.sync_copy(zbuf_v, acc.at[pl.ds(s * rpt, rpt)])
    pltpu.sync_copy(rows_hbm.at[wid], ridx_v)
    plsc.subcore_barrier()

    def body(g, carry):
      descs = [
          pltpu.async_copy(ones_v, acc.at[ridx_v.at[g * NBUF + b]], sem,
                           add=True)
          for b in range(NBUF)
      ]
      for d in descs:
        d.wait()
      return carry

    lax.fori_loop(0, nch // NBUF, body, 0)
    plsc.subcore_barrier()
    pltpu.sync_copy(acc.at[pl.ds(s * rpt, rpt)], zbuf_v)
    pltpu.sync_copy(zbuf_v, out_hbm.at[pl.ds(c * nacc + s * rpt, rpt)])

  return k


def _msg_kernel(nacc, nch, f):
  rpt = nacc // NS

  @functools.partial(
      pl.kernel,
      out_type=jax.ShapeDtypeStruct((NC, nacc, f), jnp.float32),
      mesh=_mesh(),
      scratch_types=[
          pltpu.VMEM((nch, CHUNK), jnp.int32),
          pltpu.VMEM((nch, CHUNK), jnp.int32),
          [pltpu.VMEM((CHUNK, f), jnp.float32) for _ in range(NBUF)],
          pltpu.VMEM((rpt, f), jnp.float32),
          pltpu.VMEM_SHARED((nacc, f), jnp.float32),
          [pltpu.SemaphoreType.DMA for _ in range(NBUF)],
          [pltpu.SemaphoreType.DMA for _ in range(NBUF)],
      ],
      compiler_params=pltpu.CompilerParams(use_tc_tiling_on_sc=False),
  )
  def k(rows_hbm, cols_hbm, g_hbm, zeros_hbm, out_hbm,
        ridx_v, cidx_v, msg_v, zbuf_v, acc, gsem, ssem):
    c = lax.axis_index("c")
    s = lax.axis_index("s")
    wid = c * NS + s
    pltpu.sync_copy(zeros_hbm.at[pl.ds(s * rpt, rpt)], zbuf_v)
    pltpu.sync_copy(zbuf_v, acc.at[pl.ds(s * rpt, rpt)])
    pltpu.sync_copy(rows_hbm.at[wid], ridx_v)
    pltpu.sync_copy(cols_hbm.at[wid], cidx_v)
    plsc.subcore_barrier()

    ngroups = nch // NBUF
    for b in range(NBUF):
      pltpu.async_copy(g_hbm.at[cidx_v.at[b]], msg_v[b], gsem[b])

    def group(g, fire_next):
      sdescs = []
      for b in range(NBUF):
        pltpu.make_async_copy(g_hbm.at[cidx_v.at[0]], msg_v[b],
                              gsem[b]).wait()
        sdescs.append(
            pltpu.async_copy(msg_v[b], acc.at[ridx_v.at[g * NBUF + b]],
                             ssem[b], add=True))
      for b in range(NBUF):
        sdescs[b].wait()
        if fire_next:
          pltpu.async_copy(g_hbm.at[cidx_v.at[(g + 1) * NBUF + b]],
                           msg_v[b], gsem[b])

    def body(g, carry):
      group(g, True)
      return carry

    lax.fori_loop(0, ngroups - 1, body, 0)
    group(ngroups - 1, False)
    plsc.subcore_barrier()
    pltpu.sync_copy(acc.at[pl.ds(s * rpt, rpt)], zbuf_v)
    pltpu.sync_copy(zbuf_v, out_hbm.at[c, pl.ds(s * rpt, rpt)])

  return k


def _mm_scale_body(hist_ref, x_ref, w_ref, o_ref):
  deg = 1.0 + hist_ref[:, 0] + hist_ref[:, 1]
  dinv = lax.rsqrt(deg)
  hm = jnp.dot(x_ref[...], w_ref[...], preferred_element_type=jnp.float32)
  o_ref[...] = hm * dinv[:, None]


def _layer1_body(hist_ref, s0_ref, s1_ref, g_ref, b_ref, w_ref, o_ref):
  deg = 1.0 + hist_ref[:, 0] + hist_ref[:, 1]
  dinv = lax.rsqrt(deg)
  pre = (s0_ref[...] + s1_ref[...] + g_ref[...]) * dinv[:, None] + b_ref[...]
  h1 = jnp.maximum(pre, 0.0)
  h2 = jnp.dot(h1, w_ref[...], preferred_element_type=jnp.float32)
  o_ref[...] = h2 * dinv[:, None]


def _layer2_body(hist_ref, s0_ref, s1_ref, g_ref, b_ref, o_ref):
  deg = 1.0 + hist_ref[:, 0] + hist_ref[:, 1]
  dinv = lax.rsqrt(deg)
  o = (s0_ref[...] + s1_ref[...] + g_ref[...]) * dinv[:, None] + b_ref[...]
  m = jnp.max(o, axis=1, keepdims=True)
  lse = jnp.log(jnp.sum(jnp.exp(o - m), axis=1, keepdims=True)) + m
  o_ref[...] = o - lse


def _row_call(body, nrows, blk, out_width, in_specs, out_dtype=jnp.float32):
  return pl.pallas_call(
      body,
      grid=(nrows // blk,),
      in_specs=in_specs,
      out_specs=pl.BlockSpec((blk, out_width), lambda i: (i, 0)),
      out_shape=jax.ShapeDtypeStruct((nrows, out_width), out_dtype),
  )


def kernel(x, edge_index, W1, b1, W2, b2):
  n, d = x.shape
  h = W1.shape[1]
  cdim = W2.shape[1]
  e = edge_index.shape[1]

  ew = -(-e // (NW * CHUNK * NBUF)) * CHUNK * NBUF
  epad = ew * NW
  nch = ew // CHUNK
  pad = epad - e
  rows = jnp.concatenate(
      [edge_index[0], jnp.full((pad,), n, jnp.int32)]).reshape(NW, nch, CHUNK)
  cols = jnp.concatenate(
      [edge_index[1], jnp.zeros((pad,), jnp.int32)]).reshape(NW, nch, CHUNK)

  nacc = -(-(n + 1) // 256) * 256
  blk = nacc // 16

  zh = jnp.zeros((nacc,), jnp.float32)
  z1 = jnp.zeros((nacc, h), jnp.float32)
  z2 = jnp.zeros((nacc, cdim), jnp.float32)
  x_pad = jnp.concatenate([x, jnp.zeros((nacc - n, d), x.dtype)])

  hist = _hist_kernel(nacc, nch)(rows, zh)
  hist_t = hist.reshape(NC, nacc).T

  g1 = _row_call(
      _mm_scale_body, nacc, blk, h,
      [pl.BlockSpec((blk, NC), lambda i: (i, 0)),
       pl.BlockSpec((blk, d), lambda i: (i, 0)),
       pl.BlockSpec((d, h), lambda i: (0, 0))])(hist_t, x_pad, W1)

  s1 = _msg_kernel(nacc, nch, h)(rows, cols, g1, z1)

  g2 = _row_call(
      _layer1_body, nacc, blk, cdim,
      [pl.BlockSpec((blk, NC), lambda i: (i, 0)),
       pl.BlockSpec((blk, h), lambda i: (i, 0)),
       pl.BlockSpec((blk, h), lambda i: (i, 0)),
       pl.BlockSpec((blk, h), lambda i: (i, 0)),
       pl.BlockSpec((1, h), lambda i: (0, 0)),
       pl.BlockSpec((h, cdim), lambda i: (0, 0))])(
           hist_t, s1[0], s1[1], g1, b1[None, :], W2)

  s2 = _msg_kernel(nacc, nch, cdim)(rows, cols, g2, z2)

  out = _row_call(
      _layer2_body, nacc, blk, cdim,
      [pl.BlockSpec((blk, NC), lambda i: (i, 0)),
       pl.BlockSpec((blk, cdim), lambda i: (i, 0)),
       pl.BlockSpec((blk, cdim), lambda i: (i, 0)),
       pl.BlockSpec((blk, cdim), lambda i: (i, 0)),
       pl.BlockSpec((1, cdim), lambda i: (0, 0))])(
           hist_t, s2[0], s2[1], g2, b2[None, :])

  return out[:n]


# speedup vs baseline: 1.2697x; 1.2697x over previous
import functools

import jax
import jax.numpy as jnp
from jax import lax
from jax.experimental import pallas as pl
from jax.experimental.pallas import tpu as pltpu
from jax.experimental.pallas import tpu_sc as plsc

NC = 2
NS = 16
NW = NC * NS
CHUNK = 128
NBUF = 4


def _mesh():
  return plsc.VectorSubcoreMesh(
      core_axis_name="c", subcore_axis_name="s", num_cores=NC,
      num_subcores=NS)


def _hist_kernel(nacc, nch):
  rpt = nacc // NS

  @functools.partial(
      pl.kernel,
      out_type=jax.ShapeDtypeStruct((NC * nacc,), jnp.float32),
      mesh=_mesh(),
      scratch_types=[
          pltpu.VMEM((nch, CHUNK), jnp.int32),
          pltpu.VMEM((CHUNK,), jnp.float32),
          pltpu.VMEM((rpt,), jnp.float32),
          pltpu.VMEM_SHARED((nacc,), jnp.float32),
          pltpu.SemaphoreType.DMA,
      ],
  )
  def k(rows_hbm, zeros_hbm, out_hbm, ridx_v, ones_v, zbuf_v, acc, sem):
    c = lax.axis_index("c")
    s = lax.axis_index("s")
    wid = c * NS + s
    for i in range(CHUNK // 16):
      ones_v[pl.ds(16 * i, 16)] = jnp.ones((16,), jnp.float32)
    pltpu.sync_copy(zeros_hbm.at[pl.ds(s * rpt, rpt)], zbuf_v)
    pltpu.sync_copy(zbuf_v, acc.at[pl.ds(s * rpt, rpt)])
    pltpu.sync_copy(rows_hbm.at[wid], ridx_v)
    plsc.subcore_barrier()

    def body(g, carry):
      descs = [
          pltpu.async_copy(ones_v, acc.at[ridx_v.at[g * 4 + b]], sem,
                           add=True)
          for b in range(4)
      ]
      for d in descs:
        d.wait()
      return carry

    lax.fori_loop(0, nch // 4, body, 0)
    plsc.subcore_barrier()
    pltpu.sync_copy(acc.at[pl.ds(s * rpt, rpt)], zbuf_v)
    pltpu.sync_copy(zbuf_v, out_hbm.at[pl.ds(c * nacc + s * rpt, rpt)])

  return k


def _msg_kernel(nacc, nch, f):
  rpt = nacc // NS

  @functools.partial(
      pl.kernel,
      out_type=jax.ShapeDtypeStruct((NC, nacc, f), jnp.float32),
      mesh=_mesh(),
      scratch_types=[
          pltpu.VMEM((nch, CHUNK), jnp.int32),
          pltpu.VMEM((nch, CHUNK), jnp.int32),
          [pltpu.VMEM((CHUNK, f), jnp.float32) for _ in range(2 * NBUF)],
          pltpu.VMEM((rpt, f), jnp.float32),
          pltpu.VMEM_SHARED((nacc, f), jnp.float32),
          [pltpu.SemaphoreType.DMA for _ in range(2 * NBUF)],
          [pltpu.SemaphoreType.DMA for _ in range(2 * NBUF)],
      ],
      compiler_params=pltpu.CompilerParams(use_tc_tiling_on_sc=False),
  )
  def k(rows_hbm, cols_hbm, g_hbm, zeros_hbm, out_hbm,
        ridx_v, cidx_v, msg_v, zbuf_v, acc, gsem, ssem):
    c = lax.axis_index("c")
    s = lax.axis_index("s")
    wid = c * NS + s
    pltpu.sync_copy(zeros_hbm.at[pl.ds(s * rpt, rpt)], zbuf_v)
    pltpu.sync_copy(zbuf_v, acc.at[pl.ds(s * rpt, rpt)])
    pltpu.sync_copy(rows_hbm.at[wid], ridx_v)
    pltpu.sync_copy(cols_hbm.at[wid], cidx_v)
    plsc.subcore_barrier()

    ngroups = nch // NBUF

    def buf(phase, b):
      return phase * NBUF + b

    for b in range(NBUF):
      pltpu.async_copy(g_hbm.at[cidx_v.at[b]], msg_v[buf(0, b)],
                       gsem[buf(0, b)])

    def run_group(g, phase, fire_pred):
      cur, nxt = phase, 1 - phase
      for b in range(NBUF):
        def fire(b=b):
          pltpu.async_copy(g_hbm.at[cidx_v.at[(g + 1) * NBUF + b]],
                           msg_v[buf(nxt, b)], gsem[buf(nxt, b)])
        if fire_pred is True:
          fire()
        else:
          pl.when(fire_pred)(fire)
      sdescs = []
      for b in range(NBUF):
        pltpu.make_async_copy(g_hbm.at[cidx_v.at[0]], msg_v[buf(cur, b)],
                              gsem[buf(cur, b)]).wait()
        sdescs.append(
            pltpu.async_copy(msg_v[buf(cur, b)],
                             acc.at[ridx_v.at[g * NBUF + b]],
                             ssem[buf(cur, b)], add=True))
      for d in sdescs:
        d.wait()

    def pair(p, carry):
      run_group(2 * p, 0, True)
      run_group(2 * p + 1, 1, p < ngroups // 2 - 1)
      return carry

    lax.fori_loop(0, ngroups // 2, pair, 0)
    plsc.subcore_barrier()
    pltpu.sync_copy(acc.at[pl.ds(s * rpt, rpt)], zbuf_v)
    pltpu.sync_copy(zbuf_v, out_hbm.at[c, pl.ds(s * rpt, rpt)])

  return k


def _mm_scale_body(hist_ref, x_ref, w_ref, o_ref):
  deg = 1.0 + hist_ref[:, 0] + hist_ref[:, 1]
  dinv = lax.rsqrt(deg)
  hm = jnp.dot(x_ref[...], w_ref[...], preferred_element_type=jnp.float32)
  o_ref[...] = hm * dinv[:, None]


def _layer1_body(hist_ref, s_ref, g_ref, b_ref, w_ref, o_ref):
  deg = 1.0 + hist_ref[:, 0] + hist_ref[:, 1]
  dinv = lax.rsqrt(deg)
  pre = (s_ref[0] + s_ref[1] + g_ref[...]) * dinv[:, None] + b_ref[...]
  h1 = jnp.maximum(pre, 0.0)
  h2 = jnp.dot(h1, w_ref[...], preferred_element_type=jnp.float32)
  o_ref[...] = h2 * dinv[:, None]


def _layer2_body(hist_ref, s_ref, g_ref, b_ref, o_ref):
  deg = 1.0 + hist_ref[:, 0] + hist_ref[:, 1]
  dinv = lax.rsqrt(deg)
  o = (s_ref[0] + s_ref[1] + g_ref[...]) * dinv[:, None] + b_ref[...]
  m = jnp.max(o, axis=1, keepdims=True)
  lse = jnp.log(jnp.sum(jnp.exp(o - m), axis=1, keepdims=True)) + m
  o_ref[...] = o - lse


def _row_call(body, nrows, blk, out_width, in_specs, out_dtype=jnp.float32):
  return pl.pallas_call(
      body,
      grid=(nrows // blk,),
      in_specs=in_specs,
      out_specs=pl.BlockSpec((blk, out_width), lambda i: (i, 0)),
      out_shape=jax.ShapeDtypeStruct((nrows, out_width), out_dtype),
  )


def kernel(x, edge_index, W1, b1, W2, b2):
  n, d = x.shape
  h = W1.shape[1]
  cdim = W2.shape[1]
  e = edge_index.shape[1]

  gw = CHUNK * NBUF * 2
  ew = -(-e // (NW * gw)) * gw
  epad = ew * NW
  nch = ew // CHUNK
  pad = epad - e
  rows = jnp.concatenate(
      [edge_index[0], jnp.full((pad,), n, jnp.int32)]).reshape(NW, nch, CHUNK)
  cols = jnp.concatenate(
      [edge_index[1], jnp.zeros((pad,), jnp.int32)]).reshape(NW, nch, CHUNK)

  nacc = -(-(n + 1) // 256) * 256
  blk = nacc // 4

  zh = jnp.zeros((nacc,), jnp.float32)
  z1 = jnp.zeros((nacc, h), jnp.float32)
  z2 = jnp.zeros((nacc, cdim), jnp.float32)
  x_pad = jnp.concatenate([x, jnp.zeros((nacc - n, d), x.dtype)])

  hist = _hist_kernel(nacc, nch)(rows, zh)
  hist_t = hist.reshape(NC, nacc).T

  g1 = _row_call(
      _mm_scale_body, nacc, blk, h,
      [pl.BlockSpec((blk, NC), lambda i: (i, 0)),
       pl.BlockSpec((blk, d), lambda i: (i, 0)),
       pl.BlockSpec((d, h), lambda i: (0, 0))])(hist_t, x_pad, W1)

  s1 = _msg_kernel(nacc, nch, h)(rows, cols, g1, z1)

  g2 = _row_call(
      _layer1_body, nacc, blk, cdim,
      [pl.BlockSpec((blk, NC), lambda i: (i, 0)),
       pl.BlockSpec((NC, blk, h), lambda i: (0, i, 0)),
       pl.BlockSpec((blk, h), lambda i: (i, 0)),
       pl.BlockSpec((1, h), lambda i: (0, 0)),
       pl.BlockSpec((h, cdim), lambda i: (0, 0))])(
           hist_t, s1, g1, b1[None, :], W2)

  s2 = _msg_kernel(nacc, nch, cdim)(rows, cols, g2, z2)

  out = _row_call(
      _layer2_body, nacc, blk, cdim,
      [pl.BlockSpec((blk, NC), lambda i: (i, 0)),
       pl.BlockSpec((NC, blk, cdim), lambda i: (0, i, 0)),
       pl.BlockSpec((blk, cdim), lambda i: (i, 0)),
       pl.BlockSpec((1, cdim), lambda i: (0, 0))])(
           hist_t, s2, g2, b2[None, :])

  return out[:n]


# speedup vs baseline: 1.2935x; 1.0188x over previous
import functools

import jax
import jax.numpy as jnp
from jax import lax
from jax.experimental import pallas as pl
from jax.experimental.pallas import tpu as pltpu
from jax.experimental.pallas import tpu_sc as plsc

NC = 2
NS = 16
NW = NC * NS
CHUNK = 128
NBUF = 4


def _mesh():
  return plsc.VectorSubcoreMesh(
      core_axis_name="c", subcore_axis_name="s", num_cores=NC,
      num_subcores=NS)


def _hist_kernel(nacc, nch):
  rpt = nacc // NS

  @functools.partial(
      pl.kernel,
      out_type=jax.ShapeDtypeStruct((NC * nacc,), jnp.float32),
      mesh=_mesh(),
      scratch_types=[
          pltpu.VMEM((nch, CHUNK), jnp.int32),
          pltpu.VMEM((CHUNK,), jnp.float32),
          pltpu.VMEM((rpt,), jnp.float32),
          pltpu.VMEM_SHARED((nacc,), jnp.float32),
          pltpu.SemaphoreType.DMA,
      ],
  )
  def k(rows_hbm, zeros_hbm, out_hbm, ridx_v, ones_v, zbuf_v, acc, sem):
    c = lax.axis_index("c")
    s = lax.axis_index("s")
    wid = c * NS + s
    for i in range(CHUNK // 16):
      ones_v[pl.ds(16 * i, 16)] = jnp.ones((16,), jnp.float32)
    pltpu.sync_copy(zeros_hbm.at[pl.ds(s * rpt, rpt)], zbuf_v)
    pltpu.sync_copy(zbuf_v, acc.at[pl.ds(s * rpt, rpt)])
    pltpu.sync_copy(rows_hbm.at[wid], ridx_v)
    plsc.subcore_barrier()

    def body(g, carry):
      descs = [
          pltpu.async_copy(ones_v, acc.at[ridx_v.at[g * 4 + b]], sem,
                           add=True)
          for b in range(4)
      ]
      for d in descs:
        d.wait()
      return carry

    lax.fori_loop(0, nch // 4, body, 0)
    plsc.subcore_barrier()
    pltpu.sync_copy(acc.at[pl.ds(s * rpt, rpt)], zbuf_v)
    pltpu.sync_copy(zbuf_v, out_hbm.at[pl.ds(c * nacc + s * rpt, rpt)])

  return k


def _msg_kernel(nacc, nch, f):
  rpt = nacc // NS

  @functools.partial(
      pl.kernel,
      out_type=jax.ShapeDtypeStruct((NC, nacc, f), jnp.float32),
      mesh=_mesh(),
      scratch_types=[
          pltpu.VMEM((nch, CHUNK), jnp.int32),
          pltpu.VMEM((nch, CHUNK), jnp.int32),
          [pltpu.VMEM((CHUNK, f), jnp.float32) for _ in range(2 * NBUF)],
          pltpu.VMEM((rpt, f), jnp.float32),
          pltpu.VMEM_SHARED((nacc, f), jnp.float32),
          [pltpu.SemaphoreType.DMA for _ in range(2 * NBUF)],
          [pltpu.SemaphoreType.DMA for _ in range(2 * NBUF)],
      ],
      compiler_params=pltpu.CompilerParams(use_tc_tiling_on_sc=False),
  )
  def k(rows_hbm, cols_hbm, g_hbm, zeros_hbm, out_hbm,
        ridx_v, cidx_v, msg_v, zbuf_v, acc, gsem, ssem):
    c = lax.axis_index("c")
    s = lax.axis_index("s")
    wid = c * NS + s
    pltpu.sync_copy(zeros_hbm.at[pl.ds(s * rpt, rpt)], zbuf_v)
    pltpu.sync_copy(zbuf_v, acc.at[pl.ds(s * rpt, rpt)])
    pltpu.sync_copy(rows_hbm.at[wid], ridx_v)
    pltpu.sync_copy(cols_hbm.at[wid], cidx_v)
    plsc.subcore_barrier()

    ngroups = nch // NBUF

    def buf(phase, b):
      return phase * NBUF + b

    for b in range(NBUF):
      pltpu.async_copy(g_hbm.at[cidx_v.at[b]], msg_v[buf(0, b)],
                       gsem[buf(0, b)])

    def run_group(g, phase, fire_pred):
      cur, nxt = phase, 1 - phase
      for b in range(NBUF):
        def fire(b=b):
          pltpu.async_copy(g_hbm.at[cidx_v.at[(g + 1) * NBUF + b]],
                           msg_v[buf(nxt, b)], gsem[buf(nxt, b)])
        if fire_pred is True:
          fire()
        else:
          pl.when(fire_pred)(fire)
      sdescs = []
      for b in range(NBUF):
        pltpu.make_async_copy(g_hbm.at[cidx_v.at[0]], msg_v[buf(cur, b)],
                              gsem[buf(cur, b)]).wait()
        sdescs.append(
            pltpu.async_copy(msg_v[buf(cur, b)],
                             acc.at[ridx_v.at[g * NBUF + b]],
                             ssem[buf(cur, b)], add=True))
      for d in sdescs:
        d.wait()

    def pair(p, carry):
      run_group(2 * p, 0, True)
      run_group(2 * p + 1, 1, p < ngroups // 2 - 1)
      return carry

    lax.fori_loop(0, ngroups // 2, pair, 0)
    plsc.subcore_barrier()
    pltpu.sync_copy(acc.at[pl.ds(s * rpt, rpt)], zbuf_v)
    pltpu.sync_copy(zbuf_v, out_hbm.at[c, pl.ds(s * rpt, rpt)])

  return k


def _mm_scale_body(hist_ref, x_ref, w_ref, o_ref):
  deg = 1.0 + hist_ref[:, 0] + hist_ref[:, 1]
  dinv = lax.rsqrt(deg)
  hm = jnp.dot(x_ref[...], w_ref[...], preferred_element_type=jnp.float32)
  o_ref[...] = hm * dinv[:, None]


def _layer1_body(hist_ref, s_ref, g_ref, b_ref, w_ref, o_ref):
  deg = 1.0 + hist_ref[:, 0] + hist_ref[:, 1]
  dinv = lax.rsqrt(deg)
  pre = (s_ref[0] + s_ref[1] + g_ref[...]) * dinv[:, None] + b_ref[...]
  h1 = jnp.maximum(pre, 0.0)
  h2 = jnp.dot(h1, w_ref[...], preferred_element_type=jnp.float32)
  o_ref[...] = h2 * dinv[:, None]


def _layer2_body(hist_ref, s_ref, g_ref, b_ref, o_ref):
  deg = 1.0 + hist_ref[:, 0] + hist_ref[:, 1]
  dinv = lax.rsqrt(deg)
  o = (s_ref[0] + s_ref[1] + g_ref[...]) * dinv[:, None] + b_ref[...]
  m = jnp.max(o, axis=1, keepdims=True)
  lse = jnp.log(jnp.sum(jnp.exp(o - m), axis=1, keepdims=True)) + m
  o_ref[...] = o - lse


def _row_call(body, nrows, blk, out_width, in_specs, out_dtype=jnp.float32):
  return pl.pallas_call(
      body,
      grid=(nrows // blk,),
      in_specs=in_specs,
      out_specs=pl.BlockSpec((blk, out_width), lambda i: (i, 0)),
      out_shape=jax.ShapeDtypeStruct((nrows, out_width), out_dtype),
  )


def kernel(x, edge_index, W1, b1, W2, b2):
  n, d = x.shape
  h = W1.shape[1]
  cdim = W2.shape[1]
  e = edge_index.shape[1]

  gw = CHUNK * NBUF * 2
  ew = -(-e // (NW * gw)) * gw
  epad = ew * NW
  nch = ew // CHUNK
  pad = epad - e
  nacc = -(-(n + 1) // 256) * 256
  pad_rows = n + jnp.arange(pad, dtype=jnp.int32) % (nacc - n)
  rows = jnp.concatenate(
      [edge_index[0], pad_rows]).reshape(NW, nch, CHUNK)
  cols = jnp.concatenate(
      [edge_index[1], jnp.zeros((pad,), jnp.int32)]).reshape(NW, nch, CHUNK)

  blk = nacc // 4

  zh = jnp.zeros((nacc,), jnp.float32)
  z1 = jnp.zeros((nacc, h), jnp.float32)
  z2 = jnp.zeros((nacc, cdim), jnp.float32)
  x_pad = jnp.concatenate([x, jnp.zeros((nacc - n, d), x.dtype)])

  hist = _hist_kernel(nacc, nch)(rows, zh)
  hist_t = hist.reshape(NC, nacc).T

  g1 = _row_call(
      _mm_scale_body, nacc, blk, h,
      [pl.BlockSpec((blk, NC), lambda i: (i, 0)),
       pl.BlockSpec((blk, d), lambda i: (i, 0)),
       pl.BlockSpec((d, h), lambda i: (0, 0))])(hist_t, x_pad, W1)

  s1 = _msg_kernel(nacc, nch, h)(rows, cols, g1, z1)

  g2 = _row_call(
      _layer1_body, nacc, blk, cdim,
      [pl.BlockSpec((blk, NC), lambda i: (i, 0)),
       pl.BlockSpec((NC, blk, h), lambda i: (0, i, 0)),
       pl.BlockSpec((blk, h), lambda i: (i, 0)),
       pl.BlockSpec((1, h), lambda i: (0, 0)),
       pl.BlockSpec((h, cdim), lambda i: (0, 0))])(
           hist_t, s1, g1, b1[None, :], W2)

  s2 = _msg_kernel(nacc, nch, cdim)(rows, cols, g2, z2)

  out = _row_call(
      _layer2_body, nacc, blk, cdim,
      [pl.BlockSpec((blk, NC), lambda i: (i, 0)),
       pl.BlockSpec((NC, blk, cdim), lambda i: (0, i, 0)),
       pl.BlockSpec((blk, cdim), lambda i: (i, 0)),
       pl.BlockSpec((1, cdim), lambda i: (0, 0))])(
           hist_t, s2, g2, b2[None, :])

  return out[:n]


# speedup vs baseline: 1.6305x; 1.2605x over previous
import functools

import jax
import jax.numpy as jnp
from jax import lax
from jax.experimental import pallas as pl
from jax.experimental.pallas import tpu as pltpu
from jax.experimental.pallas import tpu_sc as plsc

NC = 2
NS = 16
NW = NC * NS
CHUNK = 128
NBUF = 4


def _mesh():
  return plsc.VectorSubcoreMesh(
      core_axis_name="c", subcore_axis_name="s", num_cores=NC,
      num_subcores=NS)


def _hist_kernel(nacc, nch):
  rpt = nacc // NS

  @functools.partial(
      pl.kernel,
      out_type=jax.ShapeDtypeStruct((NC * nacc,), jnp.float32),
      mesh=_mesh(),
      scratch_types=[
          pltpu.VMEM((nch, CHUNK), jnp.int32),
          pltpu.VMEM((CHUNK,), jnp.float32),
          pltpu.VMEM((rpt,), jnp.float32),
          pltpu.VMEM_SHARED((nacc,), jnp.float32),
          pltpu.SemaphoreType.DMA,
      ],
  )
  def k(rows_hbm, zeros_hbm, out_hbm, ridx_v, ones_v, zbuf_v, acc, sem):
    c = lax.axis_index("c")
    s = lax.axis_index("s")
    wid = c * NS + s
    for i in range(CHUNK // 16):
      ones_v[pl.ds(16 * i, 16)] = jnp.ones((16,), jnp.float32)
    pltpu.sync_copy(zeros_hbm.at[pl.ds(s * rpt, rpt)], zbuf_v)
    pltpu.sync_copy(zbuf_v, acc.at[pl.ds(s * rpt, rpt)])
    pltpu.sync_copy(rows_hbm.at[wid], ridx_v)
    plsc.subcore_barrier()

    def body(g, carry):
      descs = [
          pltpu.async_copy(ones_v, acc.at[ridx_v.at[g * 4 + b]], sem,
                           add=True)
          for b in range(4)
      ]
      for d in descs:
        d.wait()
      return carry

    lax.fori_loop(0, nch // 4, body, 0)
    plsc.subcore_barrier()
    pltpu.sync_copy(acc.at[pl.ds(s * rpt, rpt)], zbuf_v)
    pltpu.sync_copy(zbuf_v, out_hbm.at[pl.ds(c * nacc + s * rpt, rpt)])

  return k


def _msg_kernel(nacc, nch, f, n, stage):
  rpt = nacc // NS
  gpt = n // NS

  @functools.partial(
      pl.kernel,
      out_type=jax.ShapeDtypeStruct((NC, nacc, f), jnp.float32),
      mesh=_mesh(),
      scratch_types=[
          pltpu.VMEM((nch, CHUNK), jnp.int32),
          pltpu.VMEM((nch, CHUNK), jnp.int32),
          [pltpu.VMEM((CHUNK, f), jnp.float32) for _ in range(2 * NBUF)],
          pltpu.VMEM((rpt, f), jnp.float32),
          pltpu.VMEM_SHARED((nacc, f), jnp.float32),
          pltpu.VMEM_SHARED((n if stage else NS, f), jnp.float32),
          [pltpu.SemaphoreType.DMA for _ in range(2 * NBUF)],
          [pltpu.SemaphoreType.DMA for _ in range(2 * NBUF)],
      ],
      compiler_params=pltpu.CompilerParams(use_tc_tiling_on_sc=False),
  )
  def k(rows_hbm, cols_hbm, g_hbm, zeros_hbm, out_hbm,
        ridx_v, cidx_v, msg_v, zbuf_v, acc, g_sp, gsem, ssem):
    c = lax.axis_index("c")
    s = lax.axis_index("s")
    wid = c * NS + s
    if stage:
      pltpu.sync_copy(g_hbm.at[pl.ds(s * gpt, gpt)],
                      zbuf_v.at[pl.ds(0, gpt)])
      pltpu.sync_copy(zbuf_v.at[pl.ds(0, gpt)], g_sp.at[pl.ds(s * gpt, gpt)])
    gtab = g_sp if stage else g_hbm
    pltpu.sync_copy(zeros_hbm.at[pl.ds(s * rpt, rpt)], zbuf_v)
    pltpu.sync_copy(zbuf_v, acc.at[pl.ds(s * rpt, rpt)])
    pltpu.sync_copy(rows_hbm.at[wid], ridx_v)
    pltpu.sync_copy(cols_hbm.at[wid], cidx_v)
    plsc.subcore_barrier()

    ngroups = nch // NBUF

    def buf(phase, b):
      return phase * NBUF + b

    for b in range(NBUF):
      pltpu.async_copy(gtab.at[cidx_v.at[b]], msg_v[buf(0, b)],
                       gsem[buf(0, b)])

    def run_group(g, phase, fire_pred):
      cur, nxt = phase, 1 - phase
      for b in range(NBUF):
        def fire(b=b):
          pltpu.async_copy(gtab.at[cidx_v.at[(g + 1) * NBUF + b]],
                           msg_v[buf(nxt, b)], gsem[buf(nxt, b)])
        if fire_pred is True:
          fire()
        else:
          pl.when(fire_pred)(fire)
      sdescs = []
      for b in range(NBUF):
        pltpu.make_async_copy(gtab.at[cidx_v.at[0]], msg_v[buf(cur, b)],
                              gsem[buf(cur, b)]).wait()
        sdescs.append(
            pltpu.async_copy(msg_v[buf(cur, b)],
                             acc.at[ridx_v.at[g * NBUF + b]],
                             ssem[buf(cur, b)], add=True))
      for d in sdescs:
        d.wait()

    def pair(p, carry):
      run_group(2 * p, 0, True)
      run_group(2 * p + 1, 1, p < ngroups // 2 - 1)
      return carry

    lax.fori_loop(0, ngroups // 2, pair, 0)
    plsc.subcore_barrier()
    pltpu.sync_copy(acc.at[pl.ds(s * rpt, rpt)], zbuf_v)
    pltpu.sync_copy(zbuf_v, out_hbm.at[c, pl.ds(s * rpt, rpt)])

  return k


def _msg_fsplit_kernel(nacc, nch2, fh, n):
  rpt = nacc // NS
  gpt = n // NS

  @functools.partial(
      pl.kernel,
      out_type=jax.ShapeDtypeStruct((NC, nacc, fh), jnp.float32),
      mesh=_mesh(),
      scratch_types=[
          pltpu.VMEM((nch2, CHUNK), jnp.int32),
          pltpu.VMEM((nch2, CHUNK), jnp.int32),
          [pltpu.VMEM((CHUNK, fh), jnp.float32) for _ in range(2 * NBUF)],
          pltpu.VMEM((rpt, fh), jnp.float32),
          pltpu.VMEM_SHARED((nacc, fh), jnp.float32),
          pltpu.VMEM_SHARED((n, fh), jnp.float32),
          [pltpu.SemaphoreType.DMA for _ in range(2 * NBUF)],
          [pltpu.SemaphoreType.DMA for _ in range(2 * NBUF)],
      ],
      compiler_params=pltpu.CompilerParams(use_tc_tiling_on_sc=False),
  )
  def k(rows_hbm, cols_hbm, g_hbm, zeros_hbm, out_hbm,
        ridx_v, cidx_v, msg_v, zbuf_v, acc, g_sp, gsem, ssem):
    c = lax.axis_index("c")
    s = lax.axis_index("s")
    pltpu.sync_copy(g_hbm.at[pl.ds(s * gpt, gpt), pl.ds(c * fh, fh)],
                    zbuf_v.at[pl.ds(0, gpt)])
    pltpu.sync_copy(zbuf_v.at[pl.ds(0, gpt)], g_sp.at[pl.ds(s * gpt, gpt)])
    pltpu.sync_copy(zeros_hbm.at[pl.ds(s * rpt, rpt)], zbuf_v)
    pltpu.sync_copy(zbuf_v, acc.at[pl.ds(s * rpt, rpt)])
    pltpu.sync_copy(rows_hbm.at[s], ridx_v)
    pltpu.sync_copy(cols_hbm.at[s], cidx_v)
    plsc.subcore_barrier()

    ngroups = nch2 // NBUF

    def buf(phase, b):
      return phase * NBUF + b

    for b in range(NBUF):
      pltpu.async_copy(g_sp.at[cidx_v.at[b]], msg_v[buf(0, b)],
                       gsem[buf(0, b)])

    def run_group(g, phase, fire_pred):
      cur, nxt = phase, 1 - phase
      for b in range(NBUF):
        def fire(b=b):
          pltpu.async_copy(g_sp.at[cidx_v.at[(g + 1) * NBUF + b]],
                           msg_v[buf(nxt, b)], gsem[buf(nxt, b)])
        if fire_pred is True:
          fire()
        else:
          pl.when(fire_pred)(fire)
      sdescs = []
      for b in range(NBUF):
        pltpu.make_async_copy(g_sp.at[cidx_v.at[0]], msg_v[buf(cur, b)],
                              gsem[buf(cur, b)]).wait()
        sdescs.append(
            pltpu.async_copy(msg_v[buf(cur, b)],
                             acc.at[ridx_v.at[g * NBUF + b]],
                             ssem[buf(cur, b)], add=True))
      for d in sdescs:
        d.wait()

    def pair(p, carry):
      run_group(2 * p, 0, True)
      run_group(2 * p + 1, 1, p < ngroups // 2 - 1)
      return carry

    lax.fori_loop(0, ngroups // 2, pair, 0)
    plsc.subcore_barrier()
    pltpu.sync_copy(acc.at[pl.ds(s * rpt, rpt)], zbuf_v)
    pltpu.sync_copy(zbuf_v, out_hbm.at[c, pl.ds(s * rpt, rpt)])

  return k


def _mm_scale_body(hist_ref, x_ref, w_ref, o_ref):
  deg = 1.0 + hist_ref[:, 0] + hist_ref[:, 1]
  dinv = lax.rsqrt(deg)
  hm = jnp.dot(x_ref[...], w_ref[...], preferred_element_type=jnp.float32)
  o_ref[...] = hm * dinv[:, None]


def _layer1_body(hist_ref, s_ref, g_ref, b_ref, w_ref, o_ref):
  deg = 1.0 + hist_ref[:, 0] + hist_ref[:, 1]
  dinv = lax.rsqrt(deg)
  pre = (s_ref[0] + s_ref[1] + g_ref[...]) * dinv[:, None] + b_ref[...]
  h1 = jnp.maximum(pre, 0.0)
  h2 = jnp.dot(h1, w_ref[...], preferred_element_type=jnp.float32)
  o_ref[...] = h2 * dinv[:, None]


def _layer2_body(hist_ref, s_ref, g_ref, b_ref, o_ref):
  deg = 1.0 + hist_ref[:, 0] + hist_ref[:, 1]
  dinv = lax.rsqrt(deg)
  cdim = o_ref.shape[1]
  full = jnp.concatenate([s_ref[0], s_ref[1]], axis=1)
  o = ((full[:, :cdim] + g_ref[:, :cdim]) * dinv[:, None] + b_ref[...])
  m = jnp.max(o, axis=1, keepdims=True)
  lse = jnp.log(jnp.sum(jnp.exp(o - m), axis=1, keepdims=True)) + m
  o_ref[...] = o - lse


def _row_call(body, nrows, blk, out_width, in_specs, out_dtype=jnp.float32):
  return pl.pallas_call(
      body,
      grid=(nrows // blk,),
      in_specs=in_specs,
      out_specs=pl.BlockSpec((blk, out_width), lambda i: (i, 0)),
      out_shape=jax.ShapeDtypeStruct((nrows, out_width), out_dtype),
  )


def kernel(x, edge_index, W1, b1, W2, b2):
  n, d = x.shape
  h = W1.shape[1]
  cdim = W2.shape[1]
  e = edge_index.shape[1]

  gw = CHUNK * NBUF * 2
  ew = -(-e // (NW * gw)) * gw
  epad = ew * NW
  nch = ew // CHUNK
  pad = epad - e
  nacc = -(-(n + 1) // 256) * 256
  pad_rows = n + jnp.arange(pad, dtype=jnp.int32) % (nacc - n)
  rows = jnp.concatenate(
      [edge_index[0], pad_rows]).reshape(NW, nch, CHUNK)
  cols = jnp.concatenate(
      [edge_index[1], jnp.zeros((pad,), jnp.int32)]).reshape(NW, nch, CHUNK)


  fh = -(-cdim // (NC * 8)) * 8
  cp = NC * fh
  zh = jnp.zeros((nacc,), jnp.float32)
  z1 = jnp.zeros((nacc, h), jnp.float32)
  z2h = jnp.zeros((nacc, fh), jnp.float32)
  W2p = jnp.concatenate([W2, jnp.zeros((h, cp - cdim), jnp.float32)], axis=1)
  tblk = 2000

  hist = _hist_kernel(nacc, nch)(rows, zh)
  hist_t = hist.reshape(NC, nacc).T

  g1 = _row_call(
      _mm_scale_body, n, tblk, h,
      [pl.BlockSpec((tblk, NC), lambda i: (i, 0)),
       pl.BlockSpec((tblk, d), lambda i: (i, 0)),
       pl.BlockSpec((d, h), lambda i: (0, 0))])(hist_t, x, W1)

  s1 = _msg_kernel(nacc, nch, h, n, False)(rows, cols, g1, z1)

  g2 = _row_call(
      _layer1_body, n, tblk, cp,
      [pl.BlockSpec((tblk, NC), lambda i: (i, 0)),
       pl.BlockSpec((NC, tblk, h), lambda i: (0, i, 0)),
       pl.BlockSpec((tblk, h), lambda i: (i, 0)),
       pl.BlockSpec((1, h), lambda i: (0, 0)),
       pl.BlockSpec((h, cp), lambda i: (0, 0))])(
           hist_t, s1, g1, b1[None, :], W2p)

  nch2 = epad // (NS * CHUNK)
  rows_f = jnp.concatenate([edge_index[0], pad_rows]).reshape(
      NS, nch2, CHUNK)
  cols_f = jnp.concatenate([edge_index[1], jnp.zeros((pad,), jnp.int32)
                            ]).reshape(NS, nch2, CHUNK)
  s2 = _msg_fsplit_kernel(nacc, nch2, fh, n)(rows_f, cols_f, g2, z2h)

  out = _row_call(
      _layer2_body, n, tblk, cdim,
      [pl.BlockSpec((tblk, NC), lambda i: (i, 0)),
       pl.BlockSpec((NC, tblk, fh), lambda i: (0, i, 0)),
       pl.BlockSpec((tblk, cp), lambda i: (i, 0)),
       pl.BlockSpec((1, cdim), lambda i: (0, 0))])(
           hist_t, s2, g2, b2[None, :])

  return out


# speedup vs baseline: 1.9584x; 1.2011x over previous
import functools

import jax
import jax.numpy as jnp
from jax import lax
from jax.experimental import pallas as pl
from jax.experimental.pallas import tpu as pltpu
from jax.experimental.pallas import tpu_sc as plsc

NC = 2
NS = 16
NW = NC * NS
CHUNK = 128
NBUF = 4


def _mesh():
  return plsc.VectorSubcoreMesh(
      core_axis_name="c", subcore_axis_name="s", num_cores=NC,
      num_subcores=NS)


def _hist_kernel(nacc, nch):
  rpt = nacc // NS

  @functools.partial(
      pl.kernel,
      out_type=jax.ShapeDtypeStruct((NC * nacc,), jnp.float32),
      mesh=_mesh(),
      scratch_types=[
          pltpu.VMEM((nch, CHUNK), jnp.int32),
          pltpu.VMEM((CHUNK,), jnp.float32),
          pltpu.VMEM((rpt,), jnp.float32),
          pltpu.VMEM_SHARED((nacc,), jnp.float32),
          pltpu.SemaphoreType.DMA,
      ],
  )
  def k(rows_hbm, zeros_hbm, out_hbm, ridx_v, ones_v, zbuf_v, acc, sem):
    c = lax.axis_index("c")
    s = lax.axis_index("s")
    wid = c * NS + s
    for i in range(CHUNK // 16):
      ones_v[pl.ds(16 * i, 16)] = jnp.ones((16,), jnp.float32)
    pltpu.sync_copy(zeros_hbm.at[pl.ds(s * rpt, rpt)], zbuf_v)
    pltpu.sync_copy(zbuf_v, acc.at[pl.ds(s * rpt, rpt)])
    pltpu.sync_copy(rows_hbm.at[wid], ridx_v)
    plsc.subcore_barrier()

    def body(g, carry):
      descs = [
          pltpu.async_copy(ones_v, acc.at[ridx_v.at[g * 4 + b]], sem,
                           add=True)
          for b in range(4)
      ]
      for d in descs:
        d.wait()
      return carry

    lax.fori_loop(0, nch // 4, body, 0)
    plsc.subcore_barrier()
    pltpu.sync_copy(acc.at[pl.ds(s * rpt, rpt)], zbuf_v)
    pltpu.sync_copy(zbuf_v, out_hbm.at[pl.ds(c * nacc + s * rpt, rpt)])

  return k


def _msg_kernel(nacc, nch, f, n, stage):
  rpt = nacc // NS
  gpt = n // NS

  @functools.partial(
      pl.kernel,
      out_type=jax.ShapeDtypeStruct((NC, nacc, f), jnp.float32),
      mesh=_mesh(),
      scratch_types=[
          pltpu.VMEM((nch, CHUNK), jnp.int32),
          pltpu.VMEM((nch, CHUNK), jnp.int32),
          [pltpu.VMEM((CHUNK, f), jnp.float32) for _ in range(2 * NBUF)],
          pltpu.VMEM((rpt, f), jnp.float32),
          pltpu.VMEM_SHARED((nacc, f), jnp.float32),
          pltpu.VMEM_SHARED((n if stage else NS, f), jnp.float32),
          [pltpu.SemaphoreType.DMA for _ in range(2 * NBUF)],
          [pltpu.SemaphoreType.DMA for _ in range(2 * NBUF)],
      ],
      compiler_params=pltpu.CompilerParams(use_tc_tiling_on_sc=False),
  )
  def k(rows_hbm, cols_hbm, g_hbm, zeros_hbm, out_hbm,
        ridx_v, cidx_v, msg_v, zbuf_v, acc, g_sp, gsem, ssem):
    c = lax.axis_index("c")
    s = lax.axis_index("s")
    wid = c * NS + s
    if stage:
      pltpu.sync_copy(g_hbm.at[pl.ds(s * gpt, gpt)],
                      zbuf_v.at[pl.ds(0, gpt)])
      pltpu.sync_copy(zbuf_v.at[pl.ds(0, gpt)], g_sp.at[pl.ds(s * gpt, gpt)])
    gtab = g_sp if stage else g_hbm
    pltpu.sync_copy(zeros_hbm.at[pl.ds(s * rpt, rpt)], zbuf_v)
    pltpu.sync_copy(zbuf_v, acc.at[pl.ds(s * rpt, rpt)])
    pltpu.sync_copy(rows_hbm.at[wid], ridx_v)
    pltpu.sync_copy(cols_hbm.at[wid], cidx_v)
    plsc.subcore_barrier()

    ngroups = nch // NBUF

    def buf(phase, b):
      return phase * NBUF + b

    for b in range(NBUF):
      pltpu.async_copy(gtab.at[cidx_v.at[b]], msg_v[buf(0, b)],
                       gsem[buf(0, b)])

    def run_group(g, phase, fire_pred):
      cur, nxt = phase, 1 - phase
      for b in range(NBUF):
        def fire(b=b):
          pltpu.async_copy(gtab.at[cidx_v.at[(g + 1) * NBUF + b]],
                           msg_v[buf(nxt, b)], gsem[buf(nxt, b)])
        if fire_pred is True:
          fire()
        else:
          pl.when(fire_pred)(fire)
      sdescs = []
      for b in range(NBUF):
        pltpu.make_async_copy(gtab.at[cidx_v.at[0]], msg_v[buf(cur, b)],
                              gsem[buf(cur, b)]).wait()
        sdescs.append(
            pltpu.async_copy(msg_v[buf(cur, b)],
                             acc.at[ridx_v.at[g * NBUF + b]],
                             ssem[buf(cur, b)], add=True))
      for d in sdescs:
        d.wait()

    def pair(p, carry):
      run_group(2 * p, 0, True)
      run_group(2 * p + 1, 1, p < ngroups // 2 - 1)
      return carry

    lax.fori_loop(0, ngroups // 2, pair, 0)
    plsc.subcore_barrier()
    pltpu.sync_copy(acc.at[pl.ds(s * rpt, rpt)], zbuf_v)
    pltpu.sync_copy(zbuf_v, out_hbm.at[c, pl.ds(s * rpt, rpt)])

  return k


def _msg_fsplit_kernel(nacc, nch2, fh, n):
  rpt = nacc // NS
  gpt = n // NS

  @functools.partial(
      pl.kernel,
      out_type=jax.ShapeDtypeStruct((NC, nacc, fh), jnp.float32),
      mesh=_mesh(),
      scratch_types=[
          pltpu.VMEM((nch2, CHUNK), jnp.int32),
          pltpu.VMEM((nch2, CHUNK), jnp.int32),
          [pltpu.VMEM((CHUNK, fh), jnp.float32) for _ in range(2 * NBUF)],
          pltpu.VMEM((rpt, fh), jnp.float32),
          pltpu.VMEM_SHARED((nacc, fh), jnp.float32),
          pltpu.VMEM_SHARED((n, fh), jnp.float32),
          [pltpu.SemaphoreType.DMA for _ in range(2 * NBUF)],
          [pltpu.SemaphoreType.DMA for _ in range(2 * NBUF)],
      ],
      compiler_params=pltpu.CompilerParams(use_tc_tiling_on_sc=False),
  )
  def k(rows_hbm, cols_hbm, g_hbm, zeros_hbm, out_hbm,
        ridx_v, cidx_v, msg_v, zbuf_v, acc, g_sp, gsem, ssem):
    c = lax.axis_index("c")
    s = lax.axis_index("s")
    pltpu.sync_copy(g_hbm.at[pl.ds(s * gpt, gpt), pl.ds(c * fh, fh)],
                    zbuf_v.at[pl.ds(0, gpt)])
    pltpu.sync_copy(zbuf_v.at[pl.ds(0, gpt)], g_sp.at[pl.ds(s * gpt, gpt)])
    pltpu.sync_copy(zeros_hbm.at[pl.ds(s * rpt, rpt)], zbuf_v)
    pltpu.sync_copy(zbuf_v, acc.at[pl.ds(s * rpt, rpt)])
    pltpu.sync_copy(rows_hbm.at[s], ridx_v)
    pltpu.sync_copy(cols_hbm.at[s], cidx_v)
    plsc.subcore_barrier()

    ngroups = nch2 // NBUF

    def buf(phase, b):
      return phase * NBUF + b

    for b in range(NBUF):
      pltpu.async_copy(g_sp.at[cidx_v.at[b]], msg_v[buf(0, b)],
                       gsem[buf(0, b)])

    def run_group(g, phase, fire_pred):
      cur, nxt = phase, 1 - phase
      for b in range(NBUF):
        def fire(b=b):
          pltpu.async_copy(g_sp.at[cidx_v.at[(g + 1) * NBUF + b]],
                           msg_v[buf(nxt, b)], gsem[buf(nxt, b)])
        if fire_pred is True:
          fire()
        else:
          pl.when(fire_pred)(fire)
      sdescs = []
      for b in range(NBUF):
        pltpu.make_async_copy(g_sp.at[cidx_v.at[0]], msg_v[buf(cur, b)],
                              gsem[buf(cur, b)]).wait()
        sdescs.append(
            pltpu.async_copy(msg_v[buf(cur, b)],
                             acc.at[ridx_v.at[g * NBUF + b]],
                             ssem[buf(cur, b)], add=True))
      for d in sdescs:
        d.wait()

    def pair(p, carry):
      run_group(2 * p, 0, True)
      run_group(2 * p + 1, 1, p < ngroups // 2 - 1)
      return carry

    lax.fori_loop(0, ngroups // 2, pair, 0)
    plsc.subcore_barrier()
    pltpu.sync_copy(acc.at[pl.ds(s * rpt, rpt)], zbuf_v)
    pltpu.sync_copy(zbuf_v, out_hbm.at[c, pl.ds(s * rpt, rpt)])

  return k


def _mm_scale_body(hist_ref, x_ref, w_ref, o_ref):
  deg = 1.0 + hist_ref[:, 0] + hist_ref[:, 1]
  dinv = lax.rsqrt(deg)
  hm = jnp.dot(x_ref[...], w_ref[...], preferred_element_type=jnp.float32)
  o_ref[...] = hm * dinv[:, None]


def _layer1_body(hist_ref, s_ref, g_ref, b_ref, w_ref, o_ref):
  deg = 1.0 + hist_ref[:, 0] + hist_ref[:, 1]
  dinv = lax.rsqrt(deg)
  full = jnp.concatenate([s_ref[0], s_ref[1]], axis=1)
  pre = (full + g_ref[...]) * dinv[:, None] + b_ref[...]
  h1 = jnp.maximum(pre, 0.0)
  h2 = jnp.dot(h1, w_ref[...], preferred_element_type=jnp.float32)
  o_ref[...] = h2 * dinv[:, None]


def _layer2_body(hist_ref, s_ref, g_ref, b_ref, o_ref):
  deg = 1.0 + hist_ref[:, 0] + hist_ref[:, 1]
  dinv = lax.rsqrt(deg)
  cdim = o_ref.shape[1]
  full = jnp.concatenate([s_ref[0], s_ref[1]], axis=1)
  o = ((full[:, :cdim] + g_ref[:, :cdim]) * dinv[:, None] + b_ref[...])
  m = jnp.max(o, axis=1, keepdims=True)
  lse = jnp.log(jnp.sum(jnp.exp(o - m), axis=1, keepdims=True)) + m
  o_ref[...] = o - lse


def _row_call(body, nrows, blk, out_width, in_specs, out_dtype=jnp.float32):
  return pl.pallas_call(
      body,
      grid=(nrows // blk,),
      in_specs=in_specs,
      out_specs=pl.BlockSpec((blk, out_width), lambda i: (i, 0)),
      out_shape=jax.ShapeDtypeStruct((nrows, out_width), out_dtype),
  )


def kernel(x, edge_index, W1, b1, W2, b2):
  n, d = x.shape
  h = W1.shape[1]
  cdim = W2.shape[1]
  e = edge_index.shape[1]

  gw = CHUNK * NBUF * 2
  ew = -(-e // (NW * gw)) * gw
  epad = ew * NW
  nch = ew // CHUNK
  pad = epad - e
  nacc = -(-(n + 1) // 256) * 256
  pad_rows = n + jnp.arange(pad, dtype=jnp.int32) % (nacc - n)
  rows = jnp.concatenate(
      [edge_index[0], pad_rows]).reshape(NW, nch, CHUNK)
  cols = jnp.concatenate(
      [edge_index[1], jnp.zeros((pad,), jnp.int32)]).reshape(NW, nch, CHUNK)


  fh = -(-cdim // (NC * 8)) * 8
  cp = NC * fh
  zh = jnp.zeros((nacc,), jnp.float32)
  z2h = jnp.zeros((nacc, fh), jnp.float32)
  W2p = jnp.concatenate([W2, jnp.zeros((h, cp - cdim), jnp.float32)], axis=1)
  tblk = 2000

  hist = _hist_kernel(nacc, nch)(rows, zh)
  hist_t = hist.reshape(NC, nacc).T

  g1 = _row_call(
      _mm_scale_body, n, tblk, h,
      [pl.BlockSpec((tblk, NC), lambda i: (i, 0)),
       pl.BlockSpec((tblk, d), lambda i: (i, 0)),
       pl.BlockSpec((d, h), lambda i: (0, 0))])(hist_t, x, W1)

  fh1 = h // NC
  nch2 = epad // (NS * CHUNK)
  rows_f = jnp.concatenate([edge_index[0], pad_rows]).reshape(
      NS, nch2, CHUNK)
  cols_f = jnp.concatenate([edge_index[1], jnp.zeros((pad,), jnp.int32)
                            ]).reshape(NS, nch2, CHUNK)
  z1h = jnp.zeros((nacc, fh1), jnp.float32)
  s1 = _msg_fsplit_kernel(nacc, nch2, fh1, n)(rows_f, cols_f, g1, z1h)

  g2 = _row_call(
      _layer1_body, n, tblk, cp,
      [pl.BlockSpec((tblk, NC), lambda i: (i, 0)),
       pl.BlockSpec((NC, tblk, fh1), lambda i: (0, i, 0)),
       pl.BlockSpec((tblk, h), lambda i: (i, 0)),
       pl.BlockSpec((1, h), lambda i: (0, 0)),
       pl.BlockSpec((h, cp), lambda i: (0, 0))])(
           hist_t, s1, g1, b1[None, :], W2p)

  s2 = _msg_fsplit_kernel(nacc, nch2, fh, n)(rows_f, cols_f, g2, z2h)

  out = _row_call(
      _layer2_body, n, tblk, cdim,
      [pl.BlockSpec((tblk, NC), lambda i: (i, 0)),
       pl.BlockSpec((NC, tblk, fh), lambda i: (0, i, 0)),
       pl.BlockSpec((tblk, cp), lambda i: (i, 0)),
       pl.BlockSpec((1, cdim), lambda i: (0, 0))])(
           hist_t, s2, g2, b2[None, :])

  return out
